# baseline (speedup 1.0000x reference)
import jax
import jax.numpy as jnp
from jax import lax
from jax.experimental import pallas as pl
from jax.experimental.pallas import tpu as pltpu
from jax.experimental.pallas import tpu_sc as plsc

NUM_BUCKETS = 100000
VOCAB = 1000
SOFTCAP = 30.0
N_TOK = 1024 * 50
SEQ = 50

NC, NS, L = 2, 16, 16
NW = NC * NS
ROWS_PER_W = N_TOK // NW
CHUNK = 32
N_CHUNKS = ROWS_PER_W // CHUNK

_PAD = -3e38


def _sc_body(ids_hbm, tgt_hbm, w_hbm, sums_hbm, ctsl_hbm,
             ids_v, tgt_v, idx_v, par_v, rows_v, sums_v, ctsl_v, sem):
    wid = lax.axis_index("s") * NC + lax.axis_index("c")
    base = wid * ROWS_PER_W

    pltpu.sync_copy(ids_hbm.at[pl.ds(base, ROWS_PER_W)], ids_v)
    pltpu.sync_copy(tgt_hbm.at[pl.ds(base, ROWS_PER_W)],
                    tgt_v.at[pl.ds(0, ROWS_PER_W)])

    iota = lax.broadcasted_iota(jnp.int32, (L,), 0)

    def hash_block(i, _):
        l0 = i * L
        prev1 = ids_v[pl.ds(l0, L)]
        lidx = l0 + iota
        prev2 = plsc.load_gather(ids_v, [jnp.maximum(lidx - 1, 0)])
        s_pos = (base + lidx) % SEQ
        prev2 = jnp.where(s_pos == 0, 0, prev2)
        h = ((prev2 * 1000003) ^ (prev1 * 92821)) % NUM_BUCKETS
        idx_v[pl.ds(l0, L)] = jnp.right_shift(h, 1)
        par_v[pl.ds(l0, L)] = jnp.bitwise_and(h, 1)
        return 0

    lax.fori_loop(0, ROWS_PER_W // L, hash_block, 0, unroll=4)

    k15 = jnp.float32(1.0 / 15.0)

    def softexp(x):
        u = jnp.exp(x * k15)
        c = jnp.float32(30.0) - jnp.float32(60.0) / (u + jnp.float32(1.0))
        return jnp.exp(c)

    def decode(w):
        lo = plsc.bitcast(jnp.left_shift(w, 16), jnp.float32)
        hi = plsc.bitcast(
            jnp.bitwise_and(w, jnp.int32(-65536)), jnp.float32)
        return lo, hi

    def start_gather(k, slot):
        return pltpu.make_async_copy(
            w_hbm.at[idx_v.at[pl.ds(k * CHUNK, CHUNK)]],
            rows_v.at[slot], sem)

    start_gather(0, 0).start()

    def chunk_body(k, _):
        slot = lax.rem(k, 2)

        @pl.when(k + 1 < N_CHUNKS)
        def _():
            start_gather(k + 1, 1 - slot).start()

        pltpu.make_async_copy(
            w_hbm.at[idx_v.at[pl.ds(k * CHUNK, CHUNK)]],
            rows_v.at[slot], sem).wait()

        for g in range(CHUNK // L):
            lrow = k * CHUNK + g * L

            def tok_body(j, sums16):
                t = g * L + j
                lt = lrow + j
                sb = par_v[pl.ds(lt, L)][0] * 4
                acc1 = jnp.zeros((L,), jnp.float32)
                acc2 = jnp.zeros((L,), jnp.float32)
                for p in range(4):
                    for h in range(8):
                        w = rows_v[slot, t, sb + p, pl.ds(h * L, L)]
                        lo, hi = decode(w)
                        acc1 = acc1 + softexp(lo)
                        acc2 = acc2 + softexp(hi)
                total = jnp.sum(acc1 + acc2)

                tg = tgt_v[pl.ds(lt, L)][0]
                pt = jnp.right_shift(tg, 8)
                hg = jnp.bitwise_and(jnp.right_shift(tg, 4), 7)
                ctsl_v[pl.ds(lt * L, L)] = rows_v[slot, t, sb + pt,
                                                  pl.ds(hg * L, L)]
                return jnp.where(iota == j, total, sums16)

            sums16 = lax.fori_loop(0, L, tok_body,
                                   jnp.zeros((L,), jnp.float32))
            sums_v[pl.ds(lrow, L)] = sums16
        return 0

    lax.fori_loop(0, N_CHUNKS, chunk_body, 0)

    pltpu.sync_copy(sums_v, sums_hbm.at[pl.ds(base, ROWS_PER_W)])
    pltpu.sync_copy(ctsl_v, ctsl_hbm.at[pl.ds(base * L, ROWS_PER_W * L)])


@jax.jit
def _sc_gather_ce(ids_i32, tgt_i32, Wi):
    with jax.enable_x64(False):
        return _sc_gather_ce_x32(ids_i32, tgt_i32, Wi)


def _sc_gather_ce_x32(ids_i32, tgt_i32, Wi):
    mesh = plsc.VectorSubcoreMesh(core_axis_name="c", subcore_axis_name="s")
    f = pl.kernel(
        _sc_body,
        out_type=(
            jax.ShapeDtypeStruct((N_TOK,), jnp.float32),
            jax.ShapeDtypeStruct((N_TOK * L,), jnp.int32),
        ),
        mesh=mesh,
        scratch_types=[
            pltpu.VMEM((ROWS_PER_W,), jnp.int32),
            pltpu.VMEM((ROWS_PER_W + L,), jnp.int32),
            pltpu.VMEM((ROWS_PER_W,), jnp.int32),
            pltpu.VMEM((ROWS_PER_W + L,), jnp.int32),
            pltpu.VMEM((2, CHUNK, 8, 128), jnp.int32),
            pltpu.VMEM((ROWS_PER_W,), jnp.float32),
            pltpu.VMEM((ROWS_PER_W * L,), jnp.int32),
            pltpu.SemaphoreType.DMA,
        ],
        compiler_params=pltpu.CompilerParams(needs_layout_passes=False),
    )
    return f(ids_i32, tgt_i32, Wi)


def _reduce_body(s_ref, ct_ref, sel_ref, half_ref, o_ref):
    r = lax.broadcasted_iota(jnp.int32, (1600, 512), 0)
    c = lax.broadcasted_iota(jnp.int32, (1600, 512), 1)
    tok = jnp.right_shift(r * 512 + c, 4)
    m0 = (tok % SEQ) == 0
    bits = jnp.where(half_ref[...] == 1,
                     jnp.bitwise_and(ct_ref[...], jnp.int32(-65536)),
                     jnp.left_shift(ct_ref[...], 16))
    ctf = lax.bitcast_convert_type(bits, jnp.float32)
    cap = jnp.float32(SOFTCAP) * jnp.tanh(ctf * jnp.float32(1.0 / SOFTCAP))
    nll = jnp.where(m0, jnp.float32(6.907755278982137),
                    jnp.log(s_ref[...]) - cap)
    o_ref[0, 0] = jnp.sum(sel_ref[...] * nll) * jnp.float32(1.0 / N_TOK)


@jax.jit
def _tc_reduce(sums, ctsl, tgt):
    lane = jnp.bitwise_and(tgt, 15)
    half = jnp.bitwise_and(jnp.right_shift(tgt, 7), 1)
    sel = (jnp.arange(L, dtype=jnp.int32)[None, :]
           == lane[:, None]).astype(jnp.float32)
    halfb = jnp.broadcast_to(half[:, None], (N_TOK, L))
    out = pl.pallas_call(
        _reduce_body,
        out_shape=jax.ShapeDtypeStruct((1, 1), jnp.float32),
        out_specs=pl.BlockSpec(memory_space=pltpu.SMEM),
    )(jnp.repeat(sums[:, None], L, 1).reshape(1600, 512),
      ctsl.reshape(1600, 512),
      sel.reshape(1600, 512),
      halfb.reshape(1600, 512).astype(jnp.int32))
    return out[0, 0]


def kernel(input_ids, target_ids, W):
    ids = input_ids.reshape(-1).astype(jnp.int32)
    tgt = target_ids.reshape(-1).astype(jnp.int32)
    wb = jnp.pad(W, ((0, 0), (0, 24)), constant_values=_PAD)
    u = lax.bitcast_convert_type(wb.astype(jnp.bfloat16), jnp.uint16)
    u = u.reshape(NUM_BUCKETS, 4, 2, 128).astype(jnp.uint32)
    wi = jnp.bitwise_or(u[:, :, 0, :],
                        jnp.left_shift(u[:, :, 1, :], 16))
    wi = wi.reshape(NUM_BUCKETS // 2, 8, 128).astype(jnp.int32)
    sums, ctsl = _sc_gather_ce(ids, tgt, wi)
    return _tc_reduce(sums, ctsl, tgt)


# speedup vs baseline: 1.9469x; 1.9469x over previous
import functools

import jax
import jax.numpy as jnp
from jax import lax
from jax.experimental import pallas as pl
from jax.experimental.pallas import tpu as pltpu
from jax.experimental.pallas import tpu_sc as plsc

NUM_BUCKETS = 100000
VOCAB = 1000
SOFTCAP = 30.0
N_TOK = 1024 * 50
SEQ = 50

NC, NS, L = 2, 16, 16
NW = NC * NS
ROWS_PER_W = N_TOK // NW
CHUNK = 32
N_CHUNKS = ROWS_PER_W // CHUNK


def _sc_body(ids_hbm, tgt_hbm, w_hbm, sums_hbm, cts_hbm,
             ids_v, tgt_v, idx_v, rows_v, sums_v, cts_v, sem):
    i32 = jnp.int32
    wid = lax.axis_index("s") * NC + lax.axis_index("c")
    base = wid * ROWS_PER_W

    pltpu.sync_copy(ids_hbm.at[pl.ds(base, ROWS_PER_W)], ids_v)
    pltpu.sync_copy(tgt_hbm.at[pl.ds(base, ROWS_PER_W)], tgt_v)

    iota = lax.broadcasted_iota(jnp.int32, (L,), 0)

    def hash_block(i, _):
        l0 = i * L
        prev1 = ids_v[pl.ds(l0, L)]
        lidx = l0 + iota
        prev2 = plsc.load_gather(ids_v, [jnp.maximum(lidx - 1, 0)])
        s_pos = (base + lidx) % SEQ
        prev2 = jnp.where(s_pos == 0, 0, prev2)
        h = ((prev2 * 1000003) ^ (prev1 * 92821)) % NUM_BUCKETS
        idx_v[pl.ds(l0, L)] = h
        return 0

    lax.fori_loop(0, ROWS_PER_W // L, hash_block, 0, unroll=4)

    inv15 = jnp.float32(1.0 / 15.0)

    def softcap(x):
        u = jnp.exp(x * inv15)
        return jnp.float32(30.0) - jnp.float32(60.0) / (u + jnp.float32(1.0))

    def start_gather(k, slot):
        return pltpu.make_async_copy(
            w_hbm.at[idx_v.at[pl.ds(k * CHUNK, CHUNK)]],
            rows_v.at[slot], sem)

    start_gather(0, 0).start()

    def chunk_body(k, _):
        slot = lax.rem(k, 2)

        @pl.when(k + 1 < N_CHUNKS)
        def _():
            start_gather(k + 1, 1 - slot).start()

        pltpu.make_async_copy(
            w_hbm.at[idx_v.at[pl.ds(k * CHUNK, CHUNK)]],
            rows_v.at[slot], sem).wait()

        for g in range(CHUNK // L):
            rows16 = g * L + iota
            lrow = k * CHUNK + g * L

            tg16 = tgt_v[pl.ds(lrow, L)]
            ct16 = softcap(plsc.load_gather(
                rows_v.at[slot],
                [rows16, jnp.right_shift(tg16, 7),
                 jnp.bitwise_and(tg16, 127)]))
            cts_v[pl.ds(lrow, L)] = ct16

            def tok_body(j, sums16):
                t = g * L + j
                acc = jnp.zeros((L,), jnp.float32)
                for s in range(8):
                    nh = 8 if s < 7 else 7
                    for h in range(nh):
                        x = rows_v[slot, t, s, pl.ds(h * L, L)]
                        e = jnp.exp(softcap(x))
                        if s == 7 and h == 6:
                            e = jnp.where(iota < 8, e, jnp.float32(0.0))
                        acc = acc + e
                total = jnp.sum(acc)
                return jnp.where(iota == j, total, sums16)

            sums16 = lax.fori_loop(0, L, tok_body,
                                   jnp.zeros((L,), jnp.float32))
            sums_v[pl.ds(lrow, L)] = sums16
        return 0

    lax.fori_loop(0, N_CHUNKS, chunk_body, 0)

    pltpu.sync_copy(sums_v, sums_hbm.at[pl.ds(base, ROWS_PER_W)])
    pltpu.sync_copy(cts_v, cts_hbm.at[pl.ds(base, ROWS_PER_W)])


@jax.jit
def _sc_gather_ce(ids_i32, tgt_i32, W):
    with jax.enable_x64(False):
        return _sc_gather_ce_x32(ids_i32, tgt_i32, W)


def _sc_gather_ce_x32(ids_i32, tgt_i32, W):
    mesh = plsc.VectorSubcoreMesh(core_axis_name="c", subcore_axis_name="s")
    f = pl.kernel(
        _sc_body,
        out_type=(
            jax.ShapeDtypeStruct((N_TOK,), jnp.float32),
            jax.ShapeDtypeStruct((N_TOK,), jnp.float32),
        ),
        mesh=mesh,
        scratch_types=[
            pltpu.VMEM((ROWS_PER_W,), jnp.int32),
            pltpu.VMEM((ROWS_PER_W,), jnp.int32),
            pltpu.VMEM((ROWS_PER_W,), jnp.int32),
            pltpu.VMEM((2, CHUNK, 8, 128), jnp.float32),
            pltpu.VMEM((ROWS_PER_W,), jnp.float32),
            pltpu.VMEM((ROWS_PER_W,), jnp.float32),
            pltpu.SemaphoreType.DMA,
        ],
        compiler_params=pltpu.CompilerParams(needs_layout_passes=False),
    )
    return f(ids_i32, tgt_i32, W)


def _reduce_body(s_ref, c_ref, o_ref):
    idx = (lax.broadcasted_iota(jnp.int32, (400, 128), 0) * 128
           + lax.broadcasted_iota(jnp.int32, (400, 128), 1))
    m0 = (idx % SEQ) == 0
    nll = jnp.where(m0, jnp.float32(6.907755278982137),
                    jnp.log(s_ref[...]) - c_ref[...])
    o_ref[0, 0] = jnp.sum(nll) * jnp.float32(1.0 / N_TOK)


@jax.jit
def _tc_reduce(sums, cts):
    out = pl.pallas_call(
        _reduce_body,
        out_shape=jax.ShapeDtypeStruct((1, 1), jnp.float32),
        out_specs=pl.BlockSpec(memory_space=pltpu.SMEM),
    )(sums.reshape(400, 128), cts.reshape(400, 128))
    return out[0, 0]


def kernel(input_ids, target_ids, W):
    ids = input_ids.reshape(-1).astype(jnp.int32)
    tgt = target_ids.reshape(-1).astype(jnp.int32)
    w3 = jnp.pad(W, ((0, 0), (0, 24))).reshape(NUM_BUCKETS, 8, 128)
    sums, cts = _sc_gather_ce(ids, tgt, w3)
    return _tc_reduce(sums, cts)


# speedup vs baseline: 1.9476x; 1.0004x over previous
import jax
import jax.numpy as jnp
from jax import lax
from jax.experimental import pallas as pl
from jax.experimental.pallas import tpu as pltpu
from jax.experimental.pallas import tpu_sc as plsc

NUM_BUCKETS = 100000
VOCAB = 1000
SOFTCAP = 30.0
N_TOK = 1024 * 50
SEQ = 50

NC, NS, L = 2, 16, 16
NW = NC * NS
ROWS_PER_W = N_TOK // NW
CHUNK = 32
N_CHUNKS = ROWS_PER_W // CHUNK


def _sc_body(ids_hbm, tgt_hbm, w_hbm, sums_hbm, cts_hbm,
             ids_v, tgt_v, idx_v, rows_v, sums_v, cts_v, sem):
    i32 = jnp.int32
    wid = lax.axis_index("s") * NC + lax.axis_index("c")
    base = wid * ROWS_PER_W

    pltpu.sync_copy(ids_hbm.at[pl.ds(base, ROWS_PER_W)], ids_v)
    pltpu.sync_copy(tgt_hbm.at[pl.ds(base, ROWS_PER_W)], tgt_v)

    iota = lax.broadcasted_iota(jnp.int32, (L,), 0)

    def hash_block(i, _):
        l0 = i * L
        prev1 = ids_v[pl.ds(l0, L)]
        lidx = l0 + iota
        prev2 = plsc.load_gather(ids_v, [jnp.maximum(lidx - 1, 0)])
        s_pos = (base + lidx) % SEQ
        prev2 = jnp.where(s_pos == 0, 0, prev2)
        h = ((prev2 * 1000003) ^ (prev1 * 92821)) % NUM_BUCKETS
        idx_v[pl.ds(l0, L)] = h
        return 0

    lax.fori_loop(0, ROWS_PER_W // L, hash_block, 0, unroll=4)

    inv15 = jnp.float32(1.0 / 15.0)

    def softcap(x):
        u = jnp.exp(x * inv15)
        return jnp.float32(30.0) - jnp.float32(60.0) / (u + jnp.float32(1.0))

    def start_gather(k, slot):
        return pltpu.make_async_copy(
            w_hbm.at[idx_v.at[pl.ds(k * CHUNK, CHUNK)]],
            rows_v.at[slot], sem)

    start_gather(0, 0).start()

    def chunk_body(k, _):
        slot = lax.rem(k, 2)

        @pl.when(k + 1 < N_CHUNKS)
        def _():
            start_gather(k + 1, 1 - slot).start()

        pltpu.make_async_copy(
            w_hbm.at[idx_v.at[pl.ds(k * CHUNK, CHUNK)]],
            rows_v.at[slot], sem).wait()

        for g in range(CHUNK // L):
            rows16 = g * L + iota
            lrow = k * CHUNK + g * L

            tg16 = tgt_v[pl.ds(lrow, L)]
            ct16 = softcap(plsc.load_gather(
                rows_v.at[slot],
                [rows16, jnp.right_shift(tg16, 7),
                 jnp.bitwise_and(tg16, 127)]))
            cts_v[pl.ds(lrow, L)] = ct16

            def tok_body(j, sums16):
                t = g * L + j
                acc = jnp.zeros((L,), jnp.float32)
                for s in range(8):
                    nh = 8 if s < 7 else 7
                    for h in range(nh):
                        x = rows_v[slot, t, s, pl.ds(h * L, L)]
                        e = jnp.exp(softcap(x))
                        if s == 7 and h == 6:
                            e = jnp.where(iota < 8, e, jnp.float32(0.0))
                        acc = acc + e
                total = jnp.sum(acc)
                return jnp.where(iota == j, total, sums16)

            sums16 = lax.fori_loop(0, L, tok_body,
                                   jnp.zeros((L,), jnp.float32))
            sums_v[pl.ds(lrow, L)] = sums16
        return 0

    lax.fori_loop(0, N_CHUNKS, chunk_body, 0)

    pltpu.sync_copy(sums_v, sums_hbm.at[pl.ds(base, ROWS_PER_W)])
    pltpu.sync_copy(cts_v, cts_hbm.at[pl.ds(base, ROWS_PER_W)])


@jax.jit
def _sc_gather_ce(ids_i32, tgt_i32, W):
    with jax.enable_x64(False):
        return _sc_gather_ce_x32(ids_i32, tgt_i32, W)


def _sc_gather_ce_x32(ids_i32, tgt_i32, W):
    mesh = plsc.VectorSubcoreMesh(core_axis_name="c", subcore_axis_name="s")
    f = pl.kernel(
        _sc_body,
        out_type=(
            jax.ShapeDtypeStruct((N_TOK,), jnp.float32),
            jax.ShapeDtypeStruct((N_TOK,), jnp.float32),
        ),
        mesh=mesh,
        scratch_types=[
            pltpu.VMEM((ROWS_PER_W,), jnp.int32),
            pltpu.VMEM((ROWS_PER_W,), jnp.int32),
            pltpu.VMEM((ROWS_PER_W,), jnp.int32),
            pltpu.VMEM((2, CHUNK, 8, 128), jnp.float32),
            pltpu.VMEM((ROWS_PER_W,), jnp.float32),
            pltpu.VMEM((ROWS_PER_W,), jnp.float32),
            pltpu.SemaphoreType.DMA,
        ],
        compiler_params=pltpu.CompilerParams(needs_layout_passes=False),
    )
    return f(ids_i32, tgt_i32, W)


def _reduce_body(s_ref, c_ref, o_ref):
    idx = (lax.broadcasted_iota(jnp.int32, (400, 128), 0) * 128
           + lax.broadcasted_iota(jnp.int32, (400, 128), 1))
    m0 = (idx % SEQ) == 0
    nll = jnp.where(m0, jnp.float32(6.907755278982137),
                    jnp.log(s_ref[...]) - c_ref[...])
    o_ref[0, 0] = jnp.sum(nll) * jnp.float32(1.0 / N_TOK)


@jax.jit
def _tc_reduce(sums, cts):
    out = pl.pallas_call(
        _reduce_body,
        out_shape=jax.ShapeDtypeStruct((1, 1), jnp.float32),
        out_specs=pl.BlockSpec(memory_space=pltpu.SMEM),
    )(sums.reshape(400, 128), cts.reshape(400, 128))
    return out[0, 0]


def kernel(input_ids, target_ids, W):
    ids = input_ids.reshape(-1).astype(jnp.int32)
    tgt = target_ids.reshape(-1).astype(jnp.int32)
    w3 = jnp.concatenate(
        [W, jnp.zeros((NUM_BUCKETS, 24), jnp.float32)],
        axis=1).reshape(NUM_BUCKETS, 8, 128)
    sums, cts = _sc_gather_ce(ids, tgt, w3)
    return _tc_reduce(sums, cts)
